# Initial kernel scaffold; baseline (speedup 1.0000x reference)
import jax
import jax.numpy as jnp
from jax.experimental import pallas as pl


def kernel(input, weight, locations):
    raise NotImplementedError("write your pallas kernel here")


# speedup vs baseline: 15.9826x; 15.9826x over previous
import jax
import jax.numpy as jnp
from jax.experimental import pallas as pl
from jax.experimental.pallas import tpu as pltpu

_B = 1024
_D = 128
_K = 1024
_EPS = 1e-6


def _som_body(x_ref, w_ref, loc_ref, locs_ref, loss_ref):
    x = x_ref[...] + _EPS
    w = w_ref[...]
    xsq = jnp.sum(x * x, axis=1, keepdims=True)
    wsq = jnp.sum(w * w, axis=0, keepdims=True)
    cross = jax.lax.dot_general(
        x, w, (((1,), (0,)), ((), ())),
        precision=jax.lax.Precision.HIGHEST,
        preferred_element_type=jnp.float32,
    )
    d2 = jnp.maximum(xsq - 2.0 * cross + wsq, 0.0)
    dists = jnp.sqrt(d2)
    mins = jnp.min(dists, axis=1, keepdims=True)
    kiota = jax.lax.broadcasted_iota(jnp.int32, (_B, _K), 1)
    idx = jnp.min(jnp.where(dists == mins, kiota, _K), axis=1, keepdims=True)
    onehot = (kiota == idx).astype(jnp.float32)
    locs_ref[...] = jax.lax.dot_general(
        onehot, loc_ref[...], (((1,), (0,)), ((), ())),
        preferred_element_type=jnp.float32,
    )
    loss_ref[...] = jnp.sum(mins, axis=0, keepdims=True) / _B


def kernel(input, weight, locations):
    locs, loss = pl.pallas_call(
        _som_body,
        out_shape=(
            jax.ShapeDtypeStruct((_B, 2), jnp.float32),
            jax.ShapeDtypeStruct((1, 1), jnp.float32),
        ),
    )(input, weight, locations)
    return locs.reshape(_B, 1, 2), loss[0, 0]
